# baseline (speedup 1.0000x reference)
import jax
import jax.numpy as jnp
from jax.experimental import pallas as pl
from jax.experimental.pallas import tpu as pltpu

_HID = 4096
_NE = 64
_MIN_K = 1
_MAX_K = 4
_MID_K = (_MIN_K + _MAX_K) // 2
_ENT_LOW = 0.3
_ENT_HIGH = 1.5
_BT = 1024


def _router_block(h_ref, w_ref, idx_ref, wgt_ref, k_ref):
    h = h_ref[...].astype(jnp.bfloat16)
    w = w_ref[...].astype(jnp.bfloat16)
    logits = jax.lax.dot_general(
        w, h, (((1,), (1,)), ((), ())), preferred_element_type=jnp.float32
    )
    logits = logits.astype(jnp.bfloat16).astype(jnp.float32)

    m = jnp.max(logits, axis=0, keepdims=True)
    lt = logits - m
    e = jnp.exp(lt)
    s = jnp.sum(e, axis=0, keepdims=True)
    entropy = jnp.log(s) - jnp.sum(e * lt, axis=0, keepdims=True) / s
    k = jnp.where(
        entropy < _ENT_LOW,
        jnp.int32(_MIN_K),
        jnp.where(entropy > _ENT_HIGH, jnp.int32(_MAX_K), jnp.int32(_MID_K)),
    )

    iota = jax.lax.broadcasted_iota(jnp.int32, e.shape, 0)
    bits = jax.lax.bitcast_convert_type(e, jnp.int32)
    keyed = (bits & ~jnp.int32(0x3F)) | (jnp.int32(_NE - 1) - iota)
    tw, ti = [], []
    for _ in range(_MAX_K):
        kj = jnp.max(keyed, axis=0, keepdims=True)
        aj = jnp.int32(_NE - 1) - (kj & jnp.int32(0x3F))
        vj = jax.lax.bitcast_convert_type(kj & ~jnp.int32(0x3F), jnp.float32)
        tw.append(vj)
        ti.append(aj)
        keyed = jnp.where(iota == aj, jnp.int32(-1), keyed)
    top_w = jnp.concatenate(tw, axis=0)
    top_i = jnp.concatenate(ti, axis=0)

    slot = jax.lax.broadcasted_iota(jnp.int32, top_w.shape, 0) < k
    mw = jnp.where(slot, top_w, 0.0)
    denom = jnp.sum(mw, axis=0, keepdims=True)
    wgt_ref[...] = (mw / denom).astype(jnp.bfloat16).reshape(1, _MAX_K, _BT)
    idx_ref[...] = jnp.where(slot, top_i, -1).reshape(1, _MAX_K, _BT)
    k_ref[...] = k.reshape(1, 1, _BT)


def kernel(hidden, W):
    T = hidden.shape[0]
    G = T // _BT
    idx, wgt, k2 = pl.pallas_call(
        _router_block,
        grid=(G,),
        in_specs=[
            pl.BlockSpec((_BT, _HID), lambda i: (i, 0)),
            pl.BlockSpec((_NE, _HID), lambda i: (0, 0)),
        ],
        out_specs=[
            pl.BlockSpec((1, _MAX_K, _BT), lambda i: (i, 0, 0)),
            pl.BlockSpec((1, _MAX_K, _BT), lambda i: (i, 0, 0)),
            pl.BlockSpec((1, 1, _BT), lambda i: (i, 0, 0)),
        ],
        out_shape=[
            jax.ShapeDtypeStruct((G, _MAX_K, _BT), jnp.int32),
            jax.ShapeDtypeStruct((G, _MAX_K, _BT), jnp.bfloat16),
            jax.ShapeDtypeStruct((G, 1, _BT), jnp.int32),
        ],
        compiler_params=pltpu.CompilerParams(
            dimension_semantics=("parallel",)
        ),
    )(hidden, W)
    idx = jnp.transpose(idx, (0, 2, 1)).reshape(T, _MAX_K)
    wgt = jnp.transpose(wgt, (0, 2, 1)).reshape(T, _MAX_K)
    return (idx, wgt, k2.reshape(T))
